# Initial kernel scaffold; baseline (speedup 1.0000x reference)
import jax
import jax.numpy as jnp
from jax.experimental import pallas as pl


def kernel(word, pos1, pos2, word_table, pos1_table, pos2_table):
    raise NotImplementedError("write your pallas kernel here")


# speedup vs baseline: 8.6157x; 8.6157x over previous
import functools

import jax
import jax.numpy as jnp
from jax import lax
from jax.experimental import pallas as pl
from jax.experimental.pallas import tpu as pltpu
from jax.experimental.pallas import tpu_sc as plsc

B = 4096
L = 200
WORD_DIM = 50
OUT_DIM = 60
PAD_DIM = 64
TOK = B * L

_info = plsc.get_sparse_core_info()
NC, NS, LANES = _info.num_cores, _info.num_subcores, _info.num_lanes
NW = NC * NS

PER_W = TOK // NW
CHUNK = 1024
NCHUNK = PER_W // CHUNK
IDX_PER_DMA = 128
NDMA = CHUNK // IDX_PER_DMA


def _sc_embed(word_pad, widx, p1idx, p2idx, p1t, p2t):
    mesh = plsc.VectorSubcoreMesh(core_axis_name="c", subcore_axis_name="s")

    @functools.partial(
        pl.kernel,
        mesh=mesh,
        out_type=jax.ShapeDtypeStruct((TOK, PAD_DIM), jnp.float32),
        compiler_params=pltpu.CompilerParams(
            needs_layout_passes=False, use_tc_tiling_on_sc=False),
        scratch_types=[
            pltpu.VMEM((CHUNK,), jnp.int32),
            pltpu.VMEM((CHUNK,), jnp.int32),
            pltpu.VMEM((CHUNK,), jnp.int32),
            pltpu.VMEM((CHUNK, PAD_DIM), jnp.float32),
            pltpu.VMEM((2 * L * 5,), jnp.float32),
            pltpu.VMEM((2 * L * 5,), jnp.float32),
            pltpu.SemaphoreType.DMA,
        ],
    )
    def k(word_hbm, widx_hbm, p1idx_hbm, p2idx_hbm, p1t_hbm, p2t_hbm,
          out_hbm, widx_v, p1idx_v, p2idx_v, rows_v, p1_v, p2_v, sem):
        wid = lax.axis_index("s") * NC + lax.axis_index("c")
        base_w = wid * PER_W
        pltpu.sync_copy(p1t_hbm, p1_v)
        pltpu.sync_copy(p2t_hbm, p2_v)

        def chunk_body(ci, carry):
            base = base_w + ci * CHUNK
            pltpu.sync_copy(widx_hbm.at[pl.ds(base, CHUNK)], widx_v)
            pltpu.sync_copy(p1idx_hbm.at[pl.ds(base, CHUNK)], p1idx_v)
            pltpu.sync_copy(p2idx_hbm.at[pl.ds(base, CHUNK)], p2idx_v)
            handles = []
            for di in range(NDMA):
                handles.append(pltpu.async_copy(
                    word_hbm.at[widx_v.at[pl.ds(di * IDX_PER_DMA,
                                                IDX_PER_DMA)]],
                    rows_v.at[pl.ds(di * IDX_PER_DMA, IDX_PER_DMA), :],
                    sem))
            for h in handles:
                h.wait()
            for g in range(CHUNK // LANES):
                t16 = lax.iota(jnp.int32, LANES) + g * LANES
                p1i = p1idx_v[pl.ds(g * LANES, LANES)] * 5
                p2i = p2idx_v[pl.ds(g * LANES, LANES)] * 5
                for j in range(5):
                    v1 = plsc.load_gather(p1_v, [p1i + j])
                    plsc.store_scatter(
                        rows_v,
                        [t16, jnp.full((LANES,), WORD_DIM + j, jnp.int32)],
                        v1)
                    v2 = plsc.load_gather(p2_v, [p2i + j])
                    plsc.store_scatter(
                        rows_v,
                        [t16, jnp.full((LANES,), WORD_DIM + 5 + j,
                                       jnp.int32)],
                        v2)
            pltpu.sync_copy(rows_v, out_hbm.at[pl.ds(base, CHUNK), :])
            return carry

        lax.fori_loop(0, NCHUNK, chunk_body, 0)

    return k(word_pad, widx, p1idx, p2idx, p1t, p2t)


def kernel(word, pos1, pos2, word_table, pos1_table, pos2_table):
    word_pad = jnp.pad(word_table, ((0, 0), (0, PAD_DIM - WORD_DIM)))
    out_pad = _sc_embed(
        word_pad,
        word.reshape(-1),
        pos1.reshape(-1),
        pos2.reshape(-1),
        pos1_table.reshape(-1),
        pos2_table.reshape(-1),
    )
    return out_pad[:, :OUT_DIM].reshape(B, L, OUT_DIM)
